# Initial kernel scaffold; baseline (speedup 1.0000x reference)
import jax
import jax.numpy as jnp
from jax.experimental import pallas as pl


def kernel(x_sample, x_gene, edge_index_expresses, edge_index_expressed_by, W_l_sg, b_l_sg, W_r_sg, W_l_gs, b_l_gs, W_r_gs, ln_g_sample, ln_b_sample, ln_g_gene, ln_b_gene):
    raise NotImplementedError("write your pallas kernel here")


# speedup vs baseline: 2.8279x; 2.8279x over previous
import functools

import jax
import jax.numpy as jnp
from jax import lax
from jax.experimental import pallas as pl
from jax.experimental.pallas import tpu as pltpu
from jax.experimental.pallas import tpu_sc as plsc

NC = 2
NS = 16
C = 128
L = 16


def _sc_mean_aggregate(x_all, src_stack, dst_stack, z128, n_dst, d, ch):
    epc = ch * C
    acc_chunks = -(-(n_dst + 1) // C)
    acc_rows = acc_chunks * C
    full = n_dst // C
    rem = n_dst % C
    assert rem % 8 == 0 and d % L == 0

    mesh = plsc.VectorSubcoreMesh(
        core_axis_name="c", subcore_axis_name="s",
        num_cores=NC, num_subcores=NS)

    @functools.partial(
        pl.kernel,
        out_type=jax.ShapeDtypeStruct((2, n_dst, d), jnp.float32),
        mesh=mesh,
        scratch_types=[
            pltpu.VMEM((C,), jnp.int32),
            pltpu.VMEM((C,), jnp.int32),
            pltpu.VMEM((C, d), jnp.float32),
            pltpu.VMEM_SHARED((acc_rows, d), jnp.float32),
            pltpu.SemaphoreType.DMA,
        ],
    )
    def body(x_all_h, src_h, dst_h, z128_h,
             out, idx_s, idx_d, rows, accum, sem):
        cid = lax.axis_index("c")
        sid = lax.axis_index("s")

        pltpu.sync_copy(z128_h, rows)
        for j in range(-(-acc_chunks // NS)):
            k = sid + NS * j
            if (j + 1) * NS <= acc_chunks:
                pltpu.sync_copy(rows, accum.at[pl.ds(k * C, C)])
            else:
                @pl.when(k < acc_chunks)
                def _():
                    pltpu.sync_copy(rows, accum.at[pl.ds(k * C, C)])
        plsc.subcore_barrier()

        def step(i, _):
            base = sid * epc + i * C
            pltpu.sync_copy(src_h.at[cid, pl.ds(base, C)], idx_s)
            pltpu.sync_copy(dst_h.at[cid, pl.ds(base, C)], idx_d)
            pltpu.async_copy(x_all_h.at[idx_s], rows, sem).wait()
            pltpu.sync_copy(rows, accum.at[idx_d], add=True)
            return 0

        lax.fori_loop(0, ch, step, 0)
        plsc.subcore_barrier()

        def copy_out(r0, nr):
            pltpu.sync_copy(accum.at[pl.ds(r0, nr)], rows.at[pl.ds(0, nr)])
            pltpu.sync_copy(rows.at[pl.ds(0, nr)], out.at[cid, pl.ds(r0, nr)])

        for j in range(-(-full // NS)):
            m = sid + NS * j
            if (j + 1) * NS <= full:
                copy_out(m * C, C)
            else:
                @pl.when(m < full)
                def _():
                    copy_out(m * C, C)
        if rem:
            @pl.when(sid == NS - 1)
            def _():
                copy_out(full * C, rem)

    return body(x_all, src_stack, dst_stack, z128)


def _tc_dense(mean, x_stack, wl, bl, wr, g, b):
    _, n, d = x_stack.shape
    bs = 1000
    nb = n // bs

    def body(m_ref, x_ref, wl_ref, bl_ref, wr_ref, g_ref, b_ref, o_ref):
        x = x_ref[0]
        h = (jnp.dot(m_ref[0], wl_ref[0], preferred_element_type=jnp.float32,
                     precision=lax.Precision.HIGHEST)
             + bl_ref[0]
             + jnp.dot(x, wr_ref[0], preferred_element_type=jnp.float32,
                       precision=lax.Precision.HIGHEST))
        t = h + x
        mu = jnp.mean(t, axis=1, keepdims=True)
        var = jnp.mean(jnp.square(t - mu), axis=1, keepdims=True)
        o_ref[0] = (t - mu) * lax.rsqrt(var + 1e-5) * g_ref[0] + b_ref[0]

    return pl.pallas_call(
        body,
        grid=(2, nb),
        in_specs=[
            pl.BlockSpec((1, bs, d), lambda i, j: (i, j, 0)),
            pl.BlockSpec((1, bs, d), lambda i, j: (i, j, 0)),
            pl.BlockSpec((1, d, d), lambda i, j: (i, 0, 0)),
            pl.BlockSpec((1, 1, d), lambda i, j: (i, 0, 0)),
            pl.BlockSpec((1, d, d), lambda i, j: (i, 0, 0)),
            pl.BlockSpec((1, 1, d), lambda i, j: (i, 0, 0)),
            pl.BlockSpec((1, 1, d), lambda i, j: (i, 0, 0)),
        ],
        out_specs=pl.BlockSpec((1, bs, d), lambda i, j: (i, j, 0)),
        out_shape=jax.ShapeDtypeStruct((2, n, d), jnp.float32),
    )(mean, x_stack, wl, bl, wr, g, b)


def kernel(x_sample, x_gene, edge_index_expresses, edge_index_expressed_by,
           W_l_sg, b_l_sg, W_r_sg, W_l_gs, b_l_gs, W_r_gs,
           ln_g_sample, ln_b_sample, ln_g_gene, ln_b_gene):
    n_s, d = x_sample.shape
    n_g = x_gene.shape[0]
    e = edge_index_expresses.shape[1]
    assert n_s == n_g and n_s % 8 == 0

    src0 = edge_index_expressed_by[0].astype(jnp.int32)
    dst0 = edge_index_expressed_by[1].astype(jnp.int32)
    src1 = edge_index_expresses[0].astype(jnp.int32) + n_g
    dst1 = edge_index_expresses[1].astype(jnp.int32)
    x_all = jnp.concatenate([x_gene, x_sample], axis=0)

    ch = -(-e // (NS * C))
    epad = NS * ch * C
    trash = n_s
    pad = epad - e
    src_stack = jnp.stack([
        jnp.pad(src0, (0, pad)),
        jnp.pad(src1, (0, pad), constant_values=n_g),
    ])
    dst_stack = jnp.stack([
        jnp.pad(dst0, (0, pad), constant_values=trash),
        jnp.pad(dst1, (0, pad), constant_values=trash),
    ])
    z128 = jnp.zeros((C, d), jnp.float32)

    mean = _sc_mean_aggregate(x_all, src_stack, dst_stack, z128, n_s, d, ch)

    cnt0 = jax.ops.segment_sum(jnp.ones((e,), jnp.float32), dst0,
                               num_segments=n_s)
    cnt1 = jax.ops.segment_sum(jnp.ones((e,), jnp.float32), dst1,
                               num_segments=n_s)
    cnt = jnp.stack([cnt0, cnt1])[:, :, None]
    mean = mean / jnp.maximum(cnt, 1.0)

    x_stack = jnp.stack([x_sample, x_gene])
    wl = jnp.stack([W_l_gs, W_l_sg])
    bl = jnp.stack([b_l_gs, b_l_sg])[:, None, :]
    wr = jnp.stack([W_r_gs, W_r_sg])
    g = jnp.stack([ln_g_sample, ln_g_gene])[:, None, :]
    b = jnp.stack([ln_b_sample, ln_b_gene])[:, None, :]
    return _tc_dense(mean, x_stack, wl, bl, wr, g, b)


# speedup vs baseline: 4.3678x; 1.5446x over previous
import functools

import jax
import jax.numpy as jnp
from jax import lax
from jax.experimental import pallas as pl
from jax.experimental.pallas import tpu as pltpu
from jax.experimental.pallas import tpu_sc as plsc

NC = 2
NS = 16
C = 128


def _sc_aggregate(x_all, src_stack, dst_stack, z128, o128, n_dst, d, ch):
    epc = ch * C
    acc_chunks = -(-(n_dst + 1) // C)
    acc_rows = acc_chunks * C
    full = n_dst // C
    rem = n_dst % C
    assert rem % 8 == 0

    mesh = plsc.VectorSubcoreMesh(
        core_axis_name="c", subcore_axis_name="s",
        num_cores=NC, num_subcores=NS)

    @functools.partial(
        pl.kernel,
        out_type=(
            jax.ShapeDtypeStruct((2, n_dst, d), jnp.float32),
            jax.ShapeDtypeStruct((2, n_dst, d), jnp.float32),
        ),
        mesh=mesh,
        scratch_types=[
            pltpu.VMEM((C,), jnp.int32),
            pltpu.VMEM((C,), jnp.int32),
            pltpu.VMEM((C, d), jnp.float32),
            pltpu.VMEM((C, d), jnp.float32),
            pltpu.VMEM_SHARED((acc_rows, d), jnp.float32),
            pltpu.SemaphoreType.DMA,
        ],
    )
    def body(x_all_h, src_h, dst_h, z128_h, o128_h,
             sum_out, cnt_out, idx_s, idx_d, rows, ones, accum, sem):
        cid = lax.axis_index("c")
        sid = lax.axis_index("s")

        def zero_accum():
            for j in range(-(-acc_chunks // NS)):
                k = sid + NS * j
                if (j + 1) * NS <= acc_chunks:
                    pltpu.sync_copy(ones, accum.at[pl.ds(k * C, C)])
                else:
                    @pl.when(k < acc_chunks)
                    def _():
                        pltpu.sync_copy(ones, accum.at[pl.ds(k * C, C)])

        def copy_out(dst_hbm):
            def chunk(r0, nr):
                pltpu.sync_copy(accum.at[pl.ds(r0, nr)], rows.at[pl.ds(0, nr)])
                pltpu.sync_copy(rows.at[pl.ds(0, nr)],
                                dst_hbm.at[cid, pl.ds(r0, nr)])

            for j in range(-(-full // NS)):
                m = sid + NS * j
                if (j + 1) * NS <= full:
                    chunk(m * C, C)
                else:
                    @pl.when(m < full)
                    def _():
                        chunk(m * C, C)
            if rem:
                @pl.when(sid == NS - 1)
                def _():
                    chunk(full * C, rem)

        pltpu.sync_copy(z128_h, ones)
        zero_accum()
        plsc.subcore_barrier()

        def step(i, _):
            base = sid * epc + i * C
            pltpu.sync_copy(src_h.at[cid, pl.ds(base, C)], idx_s)
            pltpu.sync_copy(dst_h.at[cid, pl.ds(base, C)], idx_d)
            pltpu.async_copy(x_all_h.at[idx_s], rows, sem).wait()
            pltpu.sync_copy(rows, accum.at[idx_d], add=True)
            return 0

        lax.fori_loop(0, ch, step, 0)
        plsc.subcore_barrier()
        copy_out(sum_out)
        plsc.subcore_barrier()

        zero_accum()
        plsc.subcore_barrier()
        pltpu.sync_copy(o128_h, ones)

        def step2(i, _):
            base = sid * epc + i * C
            pltpu.sync_copy(dst_h.at[cid, pl.ds(base, C)], idx_d)
            pltpu.sync_copy(ones, accum.at[idx_d], add=True)
            return 0

        lax.fori_loop(0, ch, step2, 0)
        plsc.subcore_barrier()
        copy_out(cnt_out)

    return body(x_all, src_stack, dst_stack, z128, o128)


def _tc_dense(summed, cnt, x_stack, wl, bl, wr, g, b):
    _, n, d = x_stack.shape
    bs = 1000
    nb = n // bs

    def body(s_ref, c_ref, x_ref, wl_ref, bl_ref, wr_ref, g_ref, b_ref, o_ref):
        mean = s_ref[0] / jnp.maximum(c_ref[0], 1.0)
        x = x_ref[0]
        h = (jnp.dot(mean, wl_ref[0], preferred_element_type=jnp.float32,
                     precision=lax.Precision.HIGHEST)
             + bl_ref[0]
             + jnp.dot(x, wr_ref[0], preferred_element_type=jnp.float32,
                       precision=lax.Precision.HIGHEST))
        t = h + x
        mu = jnp.mean(t, axis=1, keepdims=True)
        var = jnp.mean(jnp.square(t - mu), axis=1, keepdims=True)
        o_ref[0] = (t - mu) * lax.rsqrt(var + 1e-5) * g_ref[0] + b_ref[0]

    return pl.pallas_call(
        body,
        grid=(2, nb),
        in_specs=[
            pl.BlockSpec((1, bs, d), lambda i, j: (i, j, 0)),
            pl.BlockSpec((1, bs, d), lambda i, j: (i, j, 0)),
            pl.BlockSpec((1, bs, d), lambda i, j: (i, j, 0)),
            pl.BlockSpec((1, d, d), lambda i, j: (i, 0, 0)),
            pl.BlockSpec((1, 1, d), lambda i, j: (i, 0, 0)),
            pl.BlockSpec((1, d, d), lambda i, j: (i, 0, 0)),
            pl.BlockSpec((1, 1, d), lambda i, j: (i, 0, 0)),
            pl.BlockSpec((1, 1, d), lambda i, j: (i, 0, 0)),
        ],
        out_specs=pl.BlockSpec((1, bs, d), lambda i, j: (i, j, 0)),
        out_shape=jax.ShapeDtypeStruct((2, n, d), jnp.float32),
    )(summed, cnt, x_stack, wl, bl, wr, g, b)


def kernel(x_sample, x_gene, edge_index_expresses, edge_index_expressed_by,
           W_l_sg, b_l_sg, W_r_sg, W_l_gs, b_l_gs, W_r_gs,
           ln_g_sample, ln_b_sample, ln_g_gene, ln_b_gene):
    n_s, d = x_sample.shape
    n_g = x_gene.shape[0]
    e = edge_index_expresses.shape[1]
    assert n_s == n_g and n_s % 8 == 0

    src0 = edge_index_expressed_by[0].astype(jnp.int32)
    dst0 = edge_index_expressed_by[1].astype(jnp.int32)
    src1 = edge_index_expresses[0].astype(jnp.int32) + n_g
    dst1 = edge_index_expresses[1].astype(jnp.int32)
    x_all = jnp.concatenate([x_gene, x_sample], axis=0)

    ch = -(-e // (NS * C))
    epad = NS * ch * C
    trash = n_s
    pad = epad - e
    src_stack = jnp.stack([
        jnp.pad(src0, (0, pad)),
        jnp.pad(src1, (0, pad), constant_values=n_g),
    ])
    dst_stack = jnp.stack([
        jnp.pad(dst0, (0, pad), constant_values=trash),
        jnp.pad(dst1, (0, pad), constant_values=trash),
    ])
    z128 = jnp.zeros((C, d), jnp.float32)
    o128 = jnp.ones((C, d), jnp.float32)

    summed, cnt = _sc_aggregate(x_all, src_stack, dst_stack, z128, o128,
                                n_s, d, ch)

    x_stack = jnp.stack([x_sample, x_gene])
    wl = jnp.stack([W_l_gs, W_l_sg])
    bl = jnp.stack([b_l_gs, b_l_sg])[:, None, :]
    wr = jnp.stack([W_r_gs, W_r_sg])
    g = jnp.stack([ln_g_sample, ln_g_gene])[:, None, :]
    b = jnp.stack([ln_b_sample, ln_b_gene])[:, None, :]
    return _tc_dense(summed, cnt, x_stack, wl, bl, wr, g, b)


# speedup vs baseline: 6.2850x; 1.4389x over previous
import functools

import jax
import jax.numpy as jnp
from jax import lax
from jax.experimental import pallas as pl
from jax.experimental.pallas import tpu as pltpu
from jax.experimental.pallas import tpu_sc as plsc

NC = 2
NS = 16
C = 128


def _sc_aggregate(x_all, src_stack, dst_stack, z128, o128, n_dst, d, ch):
    epc = ch * C
    acc_chunks = -(-(n_dst + 1) // C)
    acc_rows = acc_chunks * C
    full = n_dst // C
    rem = n_dst % C
    assert rem % 8 == 0

    mesh = plsc.VectorSubcoreMesh(
        core_axis_name="c", subcore_axis_name="s",
        num_cores=NC, num_subcores=NS)

    @functools.partial(
        pl.kernel,
        out_type=(
            jax.ShapeDtypeStruct((2, n_dst, d), jnp.float32),
            jax.ShapeDtypeStruct((2, n_dst, d), jnp.float32),
        ),
        mesh=mesh,
        scratch_types=[
            pltpu.VMEM((C,), jnp.int32),
            pltpu.VMEM((C,), jnp.int32),
            pltpu.VMEM((C,), jnp.int32),
            pltpu.VMEM((C,), jnp.int32),
            pltpu.VMEM((C, d), jnp.float32),
            pltpu.VMEM((C, d), jnp.float32),
            pltpu.VMEM((C, d), jnp.float32),
            pltpu.VMEM_SHARED((acc_rows, d), jnp.float32),
            pltpu.SemaphoreType.DMA,
            pltpu.SemaphoreType.DMA,
            pltpu.SemaphoreType.DMA,
            pltpu.SemaphoreType.DMA,
        ],
    )
    def body(x_all_h, src_h, dst_h, z128_h, o128_h,
             sum_out, cnt_out, idx_s0, idx_s1, idx_d0, idx_d1,
             rows0, rows1, ones, accum, gs0, gs1, ss0, ss1):
        cid = lax.axis_index("c")
        sid = lax.axis_index("s")

        def zero_accum():
            for j in range(-(-acc_chunks // NS)):
                k = sid + NS * j
                if (j + 1) * NS <= acc_chunks:
                    pltpu.sync_copy(ones, accum.at[pl.ds(k * C, C)])
                else:
                    @pl.when(k < acc_chunks)
                    def _():
                        pltpu.sync_copy(ones, accum.at[pl.ds(k * C, C)])

        def copy_out(dst_hbm):
            def chunk(r0, nr):
                pltpu.sync_copy(accum.at[pl.ds(r0, nr)], rows0.at[pl.ds(0, nr)])
                pltpu.sync_copy(rows0.at[pl.ds(0, nr)],
                                dst_hbm.at[cid, pl.ds(r0, nr)])

            for j in range(-(-full // NS)):
                m = sid + NS * j
                if (j + 1) * NS <= full:
                    chunk(m * C, C)
                else:
                    @pl.when(m < full)
                    def _():
                        chunk(m * C, C)
            if rem:
                @pl.when(sid == NS - 1)
                def _():
                    chunk(full * C, rem)

        tbase = sid * epc

        def load_idx0(i):
            pltpu.sync_copy(src_h.at[cid, pl.ds(tbase + i * C, C)], idx_s0)
            pltpu.sync_copy(dst_h.at[cid, pl.ds(tbase + i * C, C)], idx_d0)

        def load_idx1(i):
            pltpu.sync_copy(src_h.at[cid, pl.ds(tbase + i * C, C)], idx_s1)
            pltpu.sync_copy(dst_h.at[cid, pl.ds(tbase + i * C, C)], idx_d1)

        pltpu.sync_copy(z128_h, ones)
        zero_accum()
        plsc.subcore_barrier()

        load_idx0(0)
        pltpu.async_copy(x_all_h.at[idx_s0], rows0, gs0)

        def pbody(j, _):
            load_idx1(2 * j + 1)
            pltpu.async_copy(x_all_h.at[idx_s1], rows1, gs1)
            pltpu.make_async_copy(x_all_h.at[idx_s0], rows0, gs0).wait()
            pltpu.sync_copy(rows0, accum.at[idx_d0], add=True)
            load_idx0(2 * j + 2)
            pltpu.async_copy(x_all_h.at[idx_s0], rows0, gs0)
            pltpu.make_async_copy(x_all_h.at[idx_s1], rows1, gs1).wait()
            pltpu.sync_copy(rows1, accum.at[idx_d1], add=True)
            return 0

        lax.fori_loop(0, (ch - 1) // 2, pbody, 0)
        pltpu.make_async_copy(x_all_h.at[idx_s0], rows0, gs0).wait()
        pltpu.sync_copy(rows0, accum.at[idx_d0], add=True)
        plsc.subcore_barrier()
        copy_out(sum_out)
        plsc.subcore_barrier()

        zero_accum()
        plsc.subcore_barrier()
        pltpu.sync_copy(o128_h, ones)

        def load_d0(i):
            pltpu.sync_copy(dst_h.at[cid, pl.ds(tbase + i * C, C)], idx_d0)

        def load_d1(i):
            pltpu.sync_copy(dst_h.at[cid, pl.ds(tbase + i * C, C)], idx_d1)

        load_d0(0)
        pltpu.async_copy(ones, accum.at[idx_d0], ss0, add=True)

        def pbody2(j, _):
            load_d1(2 * j + 1)
            pltpu.async_copy(ones, accum.at[idx_d1], ss1, add=True)
            pltpu.make_async_copy(ones, accum.at[idx_d0], ss0).wait()
            load_d0(2 * j + 2)
            pltpu.async_copy(ones, accum.at[idx_d0], ss0, add=True)
            pltpu.make_async_copy(ones, accum.at[idx_d1], ss1).wait()
            return 0

        lax.fori_loop(0, (ch - 1) // 2, pbody2, 0)
        pltpu.make_async_copy(ones, accum.at[idx_d0], ss0).wait()
        plsc.subcore_barrier()
        copy_out(cnt_out)

    return body(x_all, src_stack, dst_stack, z128, o128)


def _tc_dense(summed, cnt, x_stack, wl, bl, wr, g, b):
    _, n, d = x_stack.shape
    bs = 1000
    nb = n // bs

    def body(s_ref, c_ref, x_ref, wl_ref, bl_ref, wr_ref, g_ref, b_ref, o_ref):
        mean = s_ref[0] / jnp.maximum(c_ref[0], 1.0)
        x = x_ref[0]
        h = (jnp.dot(mean, wl_ref[0], preferred_element_type=jnp.float32,
                     precision=lax.Precision.HIGHEST)
             + bl_ref[0]
             + jnp.dot(x, wr_ref[0], preferred_element_type=jnp.float32,
                       precision=lax.Precision.HIGHEST))
        t = h + x
        mu = jnp.mean(t, axis=1, keepdims=True)
        var = jnp.mean(jnp.square(t - mu), axis=1, keepdims=True)
        o_ref[0] = (t - mu) * lax.rsqrt(var + 1e-5) * g_ref[0] + b_ref[0]

    return pl.pallas_call(
        body,
        grid=(2, nb),
        in_specs=[
            pl.BlockSpec((1, bs, d), lambda i, j: (i, j, 0)),
            pl.BlockSpec((1, bs, d), lambda i, j: (i, j, 0)),
            pl.BlockSpec((1, bs, d), lambda i, j: (i, j, 0)),
            pl.BlockSpec((1, d, d), lambda i, j: (i, 0, 0)),
            pl.BlockSpec((1, 1, d), lambda i, j: (i, 0, 0)),
            pl.BlockSpec((1, d, d), lambda i, j: (i, 0, 0)),
            pl.BlockSpec((1, 1, d), lambda i, j: (i, 0, 0)),
            pl.BlockSpec((1, 1, d), lambda i, j: (i, 0, 0)),
        ],
        out_specs=pl.BlockSpec((1, bs, d), lambda i, j: (i, j, 0)),
        out_shape=jax.ShapeDtypeStruct((2, n, d), jnp.float32),
    )(summed, cnt, x_stack, wl, bl, wr, g, b)


def kernel(x_sample, x_gene, edge_index_expresses, edge_index_expressed_by,
           W_l_sg, b_l_sg, W_r_sg, W_l_gs, b_l_gs, W_r_gs,
           ln_g_sample, ln_b_sample, ln_g_gene, ln_b_gene):
    n_s, d = x_sample.shape
    n_g = x_gene.shape[0]
    e = edge_index_expresses.shape[1]
    assert n_s == n_g and n_s % 8 == 0

    src0 = edge_index_expressed_by[0].astype(jnp.int32)
    dst0 = edge_index_expressed_by[1].astype(jnp.int32)
    src1 = edge_index_expresses[0].astype(jnp.int32) + n_g
    dst1 = edge_index_expresses[1].astype(jnp.int32)
    x_all = jnp.concatenate([x_gene, x_sample], axis=0)

    ch = -(-e // (NS * C))
    if ch % 2 == 0:
        ch += 1
    epad = NS * ch * C
    trash = n_s
    pad = epad - e
    src_stack = jnp.stack([
        jnp.pad(src0, (0, pad)),
        jnp.pad(src1, (0, pad), constant_values=n_g),
    ])
    dst_stack = jnp.stack([
        jnp.pad(dst0, (0, pad), constant_values=trash),
        jnp.pad(dst1, (0, pad), constant_values=trash),
    ])
    z128 = jnp.zeros((C, d), jnp.float32)
    o128 = jnp.ones((C, d), jnp.float32)

    summed, cnt = _sc_aggregate(x_all, src_stack, dst_stack, z128, o128,
                                n_s, d, ch)

    x_stack = jnp.stack([x_sample, x_gene])
    wl = jnp.stack([W_l_gs, W_l_sg])
    bl = jnp.stack([b_l_gs, b_l_sg])[:, None, :]
    wr = jnp.stack([W_r_gs, W_r_sg])
    g = jnp.stack([ln_g_sample, ln_g_gene])[:, None, :]
    b = jnp.stack([ln_b_sample, ln_b_gene])[:, None, :]
    return _tc_dense(summed, cnt, x_stack, wl, bl, wr, g, b)


# speedup vs baseline: 7.1800x; 1.1424x over previous
import functools

import jax
import jax.numpy as jnp
from jax import lax
from jax.experimental import pallas as pl
from jax.experimental.pallas import tpu as pltpu
from jax.experimental.pallas import tpu_sc as plsc

NC = 2
NS = 16
C = 128


def _sc_aggregate(x_all, src_stack, dst_stack, z128, o128, n_dst, d, ch):
    epc = ch * C
    acc_chunks = -(-(n_dst + 1) // C)
    acc_rows = acc_chunks * C
    full = n_dst // C
    rem = n_dst % C
    assert rem % 8 == 0

    mesh = plsc.VectorSubcoreMesh(
        core_axis_name="c", subcore_axis_name="s",
        num_cores=NC, num_subcores=NS)

    @functools.partial(
        pl.kernel,
        out_type=(
            jax.ShapeDtypeStruct((2, n_dst, d), jnp.float32),
            jax.ShapeDtypeStruct((2, n_dst, d), jnp.float32),
        ),
        mesh=mesh,
        scratch_types=[
            [pltpu.VMEM((C,), jnp.int32)] * 4,
            [pltpu.VMEM((C,), jnp.int32)] * 4,
            pltpu.VMEM((C, d), jnp.float32),
            pltpu.VMEM((C, d), jnp.float32),
            pltpu.VMEM_SHARED((acc_rows, d), jnp.float32),
            pltpu.SemaphoreType.DMA,
            pltpu.SemaphoreType.DMA,
            pltpu.SemaphoreType.DMA,
            pltpu.SemaphoreType.DMA,
            [pltpu.SemaphoreType.DMA] * 4,
        ],
    )
    def body(x_all_h, src_h, dst_h, z128_h, o128_h,
             sum_out, cnt_out, isl, idl,
             rows0, rows1, accum, gs0, gs1, ss0, ss1, islot):
        cid = lax.axis_index("c")
        sid = lax.axis_index("s")
        rows = (rows0, rows1)
        gsem = (gs0, gs1)
        ssem = (ss0, ss1)

        def zero_accum():
            for j in range(-(-acc_chunks // NS)):
                k = sid + NS * j
                if (j + 1) * NS <= acc_chunks:
                    pltpu.sync_copy(rows0, accum.at[pl.ds(k * C, C)])
                else:
                    @pl.when(k < acc_chunks)
                    def _():
                        pltpu.sync_copy(rows0, accum.at[pl.ds(k * C, C)])

        def copy_out(dst_hbm):
            def chunk(r0, nr):
                pltpu.sync_copy(accum.at[pl.ds(r0, nr)], rows0.at[pl.ds(0, nr)])
                pltpu.sync_copy(rows0.at[pl.ds(0, nr)],
                                dst_hbm.at[cid, pl.ds(r0, nr)])

            for j in range(-(-full // NS)):
                m = sid + NS * j
                if (j + 1) * NS <= full:
                    chunk(m * C, C)
                else:
                    @pl.when(m < full)
                    def _():
                        chunk(m * C, C)
            if rem:
                @pl.when(sid == NS - 1)
                def _():
                    chunk(full * C, rem)

        tbase = sid * epc

        def load_slot(k, i):
            pltpu.async_copy(src_h.at[cid, pl.ds(tbase + i * C, C)],
                             isl[k], islot[k])
            pltpu.async_copy(dst_h.at[cid, pl.ds(tbase + i * C, C)],
                             idl[k], islot[k])

        def wait_slot(k, src_too=True):
            if src_too:
                pltpu.make_async_copy(src_h.at[cid, pl.ds(tbase, C)],
                                      isl[k], islot[k]).wait()
            pltpu.make_async_copy(dst_h.at[cid, pl.ds(tbase, C)],
                                  idl[k], islot[k]).wait()

        def load_slot_sync(k, i):
            pltpu.sync_copy(src_h.at[cid, pl.ds(tbase + i * C, C)], isl[k])
            pltpu.sync_copy(dst_h.at[cid, pl.ds(tbase + i * C, C)], idl[k])

        pltpu.sync_copy(z128_h, rows0)
        zero_accum()
        plsc.subcore_barrier()

        for k in range(4):
            load_slot_sync(k, k)
        pltpu.async_copy(x_all_h.at[isl[0]], rows0, gs0)

        def pbody(j, _):
            c0 = 4 * j
            for m in range(4):
                c = c0 + m
                kn = (m + 1) % 4

                @pl.when(c + 1 >= 4)
                def _():
                    wait_slot(kn)
                pltpu.async_copy(x_all_h.at[isl[kn]],
                                 rows[(m + 1) % 2], gsem[(m + 1) % 2])
                pltpu.make_async_copy(x_all_h.at[isl[m % 4]],
                                      rows[m % 2], gsem[m % 2]).wait()
                pltpu.sync_copy(rows[m % 2], accum.at[idl[m % 4]], add=True)

                @pl.when(c + 4 < ch)
                def _():
                    load_slot(m % 4, c + 4)
            return 0

        lax.fori_loop(0, (ch - 1) // 4, pbody, 0)
        pltpu.make_async_copy(x_all_h.at[isl[0]], rows0, gs0).wait()
        pltpu.sync_copy(rows0, accum.at[idl[0]], add=True)
        plsc.subcore_barrier()
        copy_out(sum_out)
        plsc.subcore_barrier()

        pltpu.sync_copy(z128_h, rows0)
        zero_accum()
        plsc.subcore_barrier()
        pltpu.sync_copy(o128_h, rows1)

        for k in range(4):
            load_slot_sync(k, k)
        pltpu.async_copy(rows1, accum.at[idl[0]], ss0, add=True)

        def pbody2(j, _):
            for m in range(4):
                c = 4 * j + 1 + m
                k = (1 + m) % 4
                p = (1 + m) % 2

                @pl.when(c >= 4)
                def _():
                    wait_slot(k, src_too=False)
                pltpu.async_copy(rows1, accum.at[idl[k]], ssem[p], add=True)
                pltpu.make_async_copy(rows1, accum.at[idl[(k + 3) % 4]],
                                      ssem[1 - p]).wait()

                @pl.when(c + 3 < ch)
                def _():
                    pltpu.async_copy(
                        dst_h.at[cid, pl.ds(tbase + (c + 3) * C, C)],
                        idl[(k + 3) % 4], islot[(k + 3) % 4])
            return 0

        lax.fori_loop(0, (ch - 1) // 4, pbody2, 0)
        pltpu.make_async_copy(rows1, accum.at[idl[0]], ss0).wait()
        plsc.subcore_barrier()
        copy_out(cnt_out)

    return body(x_all, src_stack, dst_stack, z128, o128)


def _tc_dense(summed, cnt, x_stack, wl, bl, wr, g, b):
    _, n, d = x_stack.shape
    bs = 1000
    nb = n // bs

    def body(s_ref, c_ref, x_ref, wl_ref, bl_ref, wr_ref, g_ref, b_ref, o_ref):
        mean = s_ref[0] / jnp.maximum(c_ref[0], 1.0)
        x = x_ref[0]
        h = (jnp.dot(mean, wl_ref[0], preferred_element_type=jnp.float32,
                     precision=lax.Precision.HIGHEST)
             + bl_ref[0]
             + jnp.dot(x, wr_ref[0], preferred_element_type=jnp.float32,
                       precision=lax.Precision.HIGHEST))
        t = h + x
        mu = jnp.mean(t, axis=1, keepdims=True)
        var = jnp.mean(jnp.square(t - mu), axis=1, keepdims=True)
        o_ref[0] = (t - mu) * lax.rsqrt(var + 1e-5) * g_ref[0] + b_ref[0]

    return pl.pallas_call(
        body,
        grid=(2, nb),
        in_specs=[
            pl.BlockSpec((1, bs, d), lambda i, j: (i, j, 0)),
            pl.BlockSpec((1, bs, d), lambda i, j: (i, j, 0)),
            pl.BlockSpec((1, bs, d), lambda i, j: (i, j, 0)),
            pl.BlockSpec((1, d, d), lambda i, j: (i, 0, 0)),
            pl.BlockSpec((1, 1, d), lambda i, j: (i, 0, 0)),
            pl.BlockSpec((1, d, d), lambda i, j: (i, 0, 0)),
            pl.BlockSpec((1, 1, d), lambda i, j: (i, 0, 0)),
            pl.BlockSpec((1, 1, d), lambda i, j: (i, 0, 0)),
        ],
        out_specs=pl.BlockSpec((1, bs, d), lambda i, j: (i, j, 0)),
        out_shape=jax.ShapeDtypeStruct((2, n, d), jnp.float32),
    )(summed, cnt, x_stack, wl, bl, wr, g, b)


def kernel(x_sample, x_gene, edge_index_expresses, edge_index_expressed_by,
           W_l_sg, b_l_sg, W_r_sg, W_l_gs, b_l_gs, W_r_gs,
           ln_g_sample, ln_b_sample, ln_g_gene, ln_b_gene):
    n_s, d = x_sample.shape
    n_g = x_gene.shape[0]
    e = edge_index_expresses.shape[1]
    assert n_s == n_g and n_s % 8 == 0

    src0 = edge_index_expressed_by[0].astype(jnp.int32)
    dst0 = edge_index_expressed_by[1].astype(jnp.int32)
    src1 = edge_index_expresses[0].astype(jnp.int32) + n_g
    dst1 = edge_index_expresses[1].astype(jnp.int32)
    x_all = jnp.concatenate([x_gene, x_sample], axis=0)

    ch = -(-e // (NS * C))
    while ch % 4 != 1:
        ch += 1
    epad = NS * ch * C
    trash = n_s
    pad = epad - e
    src_stack = jnp.stack([
        jnp.pad(src0, (0, pad)),
        jnp.pad(src1, (0, pad), constant_values=n_g),
    ])
    dst_stack = jnp.stack([
        jnp.pad(dst0, (0, pad), constant_values=trash),
        jnp.pad(dst1, (0, pad), constant_values=trash),
    ])
    z128 = jnp.zeros((C, d), jnp.float32)
    o128 = jnp.ones((C, d), jnp.float32)

    summed, cnt = _sc_aggregate(x_all, src_stack, dst_stack, z128, o128,
                                n_s, d, ch)

    x_stack = jnp.stack([x_sample, x_gene])
    wl = jnp.stack([W_l_gs, W_l_sg])
    bl = jnp.stack([b_l_gs, b_l_sg])[:, None, :]
    wr = jnp.stack([W_r_gs, W_r_sg])
    g = jnp.stack([ln_g_sample, ln_g_gene])[:, None, :]
    b = jnp.stack([ln_b_sample, ln_b_gene])[:, None, :]
    return _tc_dense(summed, cnt, x_stack, wl, bl, wr, g, b)


# speedup vs baseline: 7.3483x; 1.0234x over previous
import functools

import jax
import jax.numpy as jnp
from jax import lax
from jax.experimental import pallas as pl
from jax.experimental.pallas import tpu as pltpu
from jax.experimental.pallas import tpu_sc as plsc

NC = 2
NS = 16
C = 128


def _sc_aggregate(x_all, src_stack, dst_stack, z128, o128, n_dst, d, ch):
    epc = ch * C
    acc_chunks = -(-(n_dst + 1) // C)
    acc_rows = acc_chunks * C
    full = n_dst // C
    rem = n_dst % C
    assert rem % 8 == 0

    mesh = plsc.VectorSubcoreMesh(
        core_axis_name="c", subcore_axis_name="s",
        num_cores=NC, num_subcores=NS)

    @functools.partial(
        pl.kernel,
        out_type=(
            jax.ShapeDtypeStruct((2, n_dst, d), jnp.float32),
            jax.ShapeDtypeStruct((2, n_dst, d), jnp.float32),
        ),
        mesh=mesh,
        scratch_types=[
            [pltpu.VMEM((C,), jnp.int32)] * 4,
            [pltpu.VMEM((C,), jnp.int32)] * 4,
            pltpu.VMEM((C, d), jnp.float32),
            pltpu.VMEM((C, d), jnp.float32),
            pltpu.VMEM_SHARED((acc_rows, d), jnp.float32),
            pltpu.SemaphoreType.DMA,
            pltpu.SemaphoreType.DMA,
            pltpu.SemaphoreType.DMA,
            pltpu.SemaphoreType.DMA,
            [pltpu.SemaphoreType.DMA] * 4,
        ],
    )
    def body(x_all_h, src_h, dst_h, z128_h, o128_h,
             sum_out, cnt_out, isl, idl,
             rows0, rows1, accum, gs0, gs1, ss0, ss1, islot):
        cid = lax.axis_index("c")
        sid = lax.axis_index("s")
        rows = (rows0, rows1)
        gsem = (gs0, gs1)
        ssem = (ss0, ss1)

        def zero_accum():
            for j in range(-(-acc_chunks // NS)):
                k = sid + NS * j
                if (j + 1) * NS <= acc_chunks:
                    pltpu.sync_copy(rows0, accum.at[pl.ds(k * C, C)])
                else:
                    @pl.when(k < acc_chunks)
                    def _():
                        pltpu.sync_copy(rows0, accum.at[pl.ds(k * C, C)])

        def copy_out(dst_hbm):
            def chunk(r0, nr):
                pltpu.sync_copy(accum.at[pl.ds(r0, nr)], rows0.at[pl.ds(0, nr)])
                pltpu.sync_copy(rows0.at[pl.ds(0, nr)],
                                dst_hbm.at[cid, pl.ds(r0, nr)])

            for j in range(-(-full // NS)):
                m = sid + NS * j
                if (j + 1) * NS <= full:
                    chunk(m * C, C)
                else:
                    @pl.when(m < full)
                    def _():
                        chunk(m * C, C)
            if rem:
                @pl.when(sid == NS - 1)
                def _():
                    chunk(full * C, rem)

        tbase = sid * epc

        def load_slot(k, i):
            pltpu.async_copy(src_h.at[cid, pl.ds(tbase + i * C, C)],
                             isl[k], islot[k])
            pltpu.async_copy(dst_h.at[cid, pl.ds(tbase + i * C, C)],
                             idl[k], islot[k])

        def wait_slot(k, src_too=True):
            if src_too:
                pltpu.make_async_copy(src_h.at[cid, pl.ds(tbase, C)],
                                      isl[k], islot[k]).wait()
            pltpu.make_async_copy(dst_h.at[cid, pl.ds(tbase, C)],
                                  idl[k], islot[k]).wait()

        def load_slot_sync(k, i):
            pltpu.sync_copy(src_h.at[cid, pl.ds(tbase + i * C, C)], isl[k])
            pltpu.sync_copy(dst_h.at[cid, pl.ds(tbase + i * C, C)], idl[k])

        pltpu.sync_copy(z128_h, rows0)
        zero_accum()
        plsc.subcore_barrier()

        for k in range(4):
            load_slot_sync(k, k)
        pltpu.async_copy(x_all_h.at[isl[0]], rows0, gs0)

        def pbody(j, _):
            c0 = 4 * j
            for m in range(4):
                c = c0 + m
                kn = (m + 1) % 4

                @pl.when(c + 1 >= 4)
                def _():
                    wait_slot(kn)
                pltpu.async_copy(x_all_h.at[isl[kn]],
                                 rows[(m + 1) % 2], gsem[(m + 1) % 2])
                pltpu.make_async_copy(x_all_h.at[isl[m % 4]],
                                      rows[m % 2], gsem[m % 2]).wait()
                pltpu.sync_copy(rows[m % 2], accum.at[idl[m % 4]], add=True)

                @pl.when(c + 4 < ch)
                def _():
                    load_slot(m % 4, c + 4)
            return 0

        lax.fori_loop(0, (ch - 1) // 4, pbody, 0)
        pltpu.make_async_copy(x_all_h.at[isl[0]], rows0, gs0).wait()
        pltpu.sync_copy(rows0, accum.at[idl[0]], add=True)
        plsc.subcore_barrier()
        copy_out(sum_out)
        plsc.subcore_barrier()

        pltpu.sync_copy(z128_h, rows0)
        zero_accum()
        plsc.subcore_barrier()
        pltpu.sync_copy(o128_h, rows1)

        for k in range(4):
            load_slot_sync(k, k)
        pltpu.async_copy(rows1, accum.at[idl[0]], ss0, add=True)

        def pbody2(j, _):
            for m in range(4):
                c = 4 * j + 1 + m
                k = (1 + m) % 4
                p = (1 + m) % 2

                @pl.when(c >= 4)
                def _():
                    wait_slot(k, src_too=False)
                pltpu.async_copy(rows1, accum.at[idl[k]], ssem[p], add=True)
                pltpu.make_async_copy(rows1, accum.at[idl[(k + 3) % 4]],
                                      ssem[1 - p]).wait()

                @pl.when(c + 3 < ch)
                def _():
                    pltpu.async_copy(
                        dst_h.at[cid, pl.ds(tbase + (c + 3) * C, C)],
                        idl[(k + 3) % 4], islot[(k + 3) % 4])
            return 0

        lax.fori_loop(0, (ch - 1) // 4, pbody2, 0)
        pltpu.make_async_copy(rows1, accum.at[idl[0]], ss0).wait()
        plsc.subcore_barrier()
        copy_out(cnt_out)

    return body(x_all, src_stack, dst_stack, z128, o128)


def _tc_dense(summed, cnt, x_all, wl, bl, wr, g, b):
    _, n, d = summed.shape
    bs = 1000
    nb = n // bs

    def body(s_ref, c_ref, x_ref, wl_ref, bl_ref, wr_ref, g_ref, b_ref, o_ref):
        mean = s_ref[0] / jnp.maximum(c_ref[0], 1.0)
        x = x_ref[...]
        h = (jnp.dot(mean, wl_ref[0], preferred_element_type=jnp.float32,
                     precision=lax.Precision.HIGHEST)
             + bl_ref[0]
             + jnp.dot(x, wr_ref[0], preferred_element_type=jnp.float32,
                       precision=lax.Precision.HIGHEST))
        t = h + x
        mu = jnp.mean(t, axis=1, keepdims=True)
        var = jnp.mean(jnp.square(t - mu), axis=1, keepdims=True)
        o_ref[0] = (t - mu) * lax.rsqrt(var + 1e-5) * g_ref[0] + b_ref[0]

    return pl.pallas_call(
        body,
        grid=(2, nb),
        in_specs=[
            pl.BlockSpec((1, bs, d), lambda i, j: (i, j, 0)),
            pl.BlockSpec((1, bs, d), lambda i, j: (i, j, 0)),
            pl.BlockSpec((bs, d), lambda i, j: ((1 - i) * nb + j, 0)),
            pl.BlockSpec((1, d, d), lambda i, j: (i, 0, 0)),
            pl.BlockSpec((1, 1, d), lambda i, j: (i, 0, 0)),
            pl.BlockSpec((1, d, d), lambda i, j: (i, 0, 0)),
            pl.BlockSpec((1, 1, d), lambda i, j: (i, 0, 0)),
            pl.BlockSpec((1, 1, d), lambda i, j: (i, 0, 0)),
        ],
        out_specs=pl.BlockSpec((1, bs, d), lambda i, j: (i, j, 0)),
        out_shape=jax.ShapeDtypeStruct((2, n, d), jnp.float32),
    )(summed, cnt, x_all, wl, bl, wr, g, b)


def kernel(x_sample, x_gene, edge_index_expresses, edge_index_expressed_by,
           W_l_sg, b_l_sg, W_r_sg, W_l_gs, b_l_gs, W_r_gs,
           ln_g_sample, ln_b_sample, ln_g_gene, ln_b_gene):
    n_s, d = x_sample.shape
    n_g = x_gene.shape[0]
    e = edge_index_expresses.shape[1]
    assert n_s == n_g and n_s % 8 == 0

    src0 = edge_index_expressed_by[0].astype(jnp.int32)
    dst0 = edge_index_expressed_by[1].astype(jnp.int32)
    src1 = edge_index_expresses[0].astype(jnp.int32) + n_g
    dst1 = edge_index_expresses[1].astype(jnp.int32)
    x_all = jnp.concatenate([x_gene, x_sample], axis=0)

    ch = -(-e // (NS * C))
    while ch % 4 != 1:
        ch += 1
    epad = NS * ch * C
    trash = n_s
    pad = epad - e
    src_stack = jnp.stack([
        jnp.pad(src0, (0, pad)),
        jnp.pad(src1, (0, pad), constant_values=n_g),
    ])
    dst_stack = jnp.stack([
        jnp.pad(dst0, (0, pad), constant_values=trash),
        jnp.pad(dst1, (0, pad), constant_values=trash),
    ])
    z128 = jnp.zeros((C, d), jnp.float32)
    o128 = jnp.ones((C, d), jnp.float32)

    summed, cnt = _sc_aggregate(x_all, src_stack, dst_stack, z128, o128,
                                n_s, d, ch)

    wl = jnp.stack([W_l_gs, W_l_sg])
    bl = jnp.stack([b_l_gs, b_l_sg])[:, None, :]
    wr = jnp.stack([W_r_gs, W_r_sg])
    g = jnp.stack([ln_g_sample, ln_g_gene])[:, None, :]
    b = jnp.stack([ln_b_sample, ln_b_gene])[:, None, :]
    return _tc_dense(summed, cnt, x_all, wl, bl, wr, g, b)


# speedup vs baseline: 7.3634x; 1.0021x over previous
import functools

import jax
import jax.numpy as jnp
from jax import lax
from jax.experimental import pallas as pl
from jax.experimental.pallas import tpu as pltpu
from jax.experimental.pallas import tpu_sc as plsc

NC = 2
NS = 16
C = 128


def _sc_aggregate(x_all, src_stack, dst_stack, z128, o128, n_dst, d, ch):
    epc = ch * C
    acc_chunks = -(-(n_dst + 1) // C)
    acc_rows = acc_chunks * C
    full = n_dst // C
    rem = n_dst % C
    assert rem % 8 == 0

    mesh = plsc.VectorSubcoreMesh(
        core_axis_name="c", subcore_axis_name="s",
        num_cores=NC, num_subcores=NS)

    @functools.partial(
        pl.kernel,
        out_type=(
            jax.ShapeDtypeStruct((2, n_dst, d), jnp.float32),
            jax.ShapeDtypeStruct((2, n_dst, d), jnp.float32),
        ),
        mesh=mesh,
        scratch_types=[
            [pltpu.VMEM((C,), jnp.int32)] * 4,
            [pltpu.VMEM((C,), jnp.int32)] * 4,
            pltpu.VMEM((C, d), jnp.float32),
            pltpu.VMEM((C, d), jnp.float32),
            pltpu.VMEM_SHARED((acc_rows, d), jnp.float32),
            pltpu.SemaphoreType.DMA,
            pltpu.SemaphoreType.DMA,
            pltpu.SemaphoreType.DMA,
            pltpu.SemaphoreType.DMA,
            [pltpu.SemaphoreType.DMA] * 4,
        ],
    )
    def body(x_all_h, src_h, dst_h, z128_h, o128_h,
             sum_out, cnt_out, isl, idl,
             rows0, rows1, accum, gs0, gs1, ss0, ss1, islot):
        cid = lax.axis_index("c")
        sid = lax.axis_index("s")
        rows = (rows0, rows1)
        gsem = (gs0, gs1)
        ssem = (ss0, ss1)

        def zero_accum():
            for j in range(-(-acc_chunks // NS)):
                k = sid + NS * j
                if (j + 1) * NS <= acc_chunks:
                    pltpu.sync_copy(rows0, accum.at[pl.ds(k * C, C)])
                else:
                    @pl.when(k < acc_chunks)
                    def _():
                        pltpu.sync_copy(rows0, accum.at[pl.ds(k * C, C)])

        def copy_out(dst_hbm, zero_after=False):
            def chunk(r0, nr):
                pltpu.sync_copy(accum.at[pl.ds(r0, nr)], rows0.at[pl.ds(0, nr)])
                pltpu.sync_copy(rows0.at[pl.ds(0, nr)],
                                dst_hbm.at[cid, pl.ds(r0, nr)])
                if zero_after:
                    pltpu.sync_copy(rows1.at[pl.ds(0, nr)],
                                    accum.at[pl.ds(r0, nr)])

            for j in range(-(-full // NS)):
                m = sid + NS * j
                if (j + 1) * NS <= full:
                    chunk(m * C, C)
                else:
                    @pl.when(m < full)
                    def _():
                        chunk(m * C, C)
            if rem:
                @pl.when(sid == NS - 1)
                def _():
                    chunk(full * C, rem)

        tbase = sid * epc

        def load_slot(k, i):
            pltpu.async_copy(src_h.at[cid, pl.ds(tbase + i * C, C)],
                             isl[k], islot[k])
            pltpu.async_copy(dst_h.at[cid, pl.ds(tbase + i * C, C)],
                             idl[k], islot[k])

        def wait_slot(k, src_too=True):
            if src_too:
                pltpu.make_async_copy(src_h.at[cid, pl.ds(tbase, C)],
                                      isl[k], islot[k]).wait()
            pltpu.make_async_copy(dst_h.at[cid, pl.ds(tbase, C)],
                                  idl[k], islot[k]).wait()

        def load_slot_sync(k, i):
            pltpu.sync_copy(src_h.at[cid, pl.ds(tbase + i * C, C)], isl[k])
            pltpu.sync_copy(dst_h.at[cid, pl.ds(tbase + i * C, C)], idl[k])

        pltpu.sync_copy(z128_h, rows0)
        zero_accum()
        plsc.subcore_barrier()

        for k in range(4):
            load_slot_sync(k, k)
        pltpu.async_copy(x_all_h.at[isl[0]], rows0, gs0)

        def pbody(j, _):
            c0 = 4 * j
            for m in range(4):
                c = c0 + m
                kn = (m + 1) % 4
                b, bn = m % 2, (m + 1) % 2

                @pl.when(c + 1 >= 4)
                def _():
                    wait_slot(kn)

                @pl.when(c >= 1)
                def _():
                    pltpu.make_async_copy(rows[bn], accum.at[idl[(m + 3) % 4]],
                                          ssem[bn]).wait()

                @pl.when(jnp.logical_and(c >= 1, c + 3 < ch))
                def _():
                    load_slot((m + 3) % 4, c + 3)
                pltpu.async_copy(x_all_h.at[isl[kn]], rows[bn], gsem[bn])
                pltpu.make_async_copy(x_all_h.at[isl[m]],
                                      rows[b], gsem[b]).wait()
                pltpu.async_copy(rows[b], accum.at[idl[m]], ssem[b], add=True)
            return 0

        lax.fori_loop(0, (ch - 1) // 4, pbody, 0)
        pltpu.make_async_copy(rows1, accum.at[idl[3]], ss1).wait()
        pltpu.make_async_copy(x_all_h.at[isl[0]], rows0, gs0).wait()
        pltpu.async_copy(rows0, accum.at[idl[0]], ss0, add=True)
        pltpu.make_async_copy(rows0, accum.at[idl[0]], ss0).wait()
        pltpu.sync_copy(z128_h, rows1)
        plsc.subcore_barrier()
        copy_out(sum_out, zero_after=True)
        plsc.subcore_barrier()

        pltpu.sync_copy(o128_h, rows1)

        for k in range(4):
            load_slot_sync(k, k)
        pltpu.async_copy(rows1, accum.at[idl[0]], ss0, add=True)

        def pbody2(j, _):
            for m in range(4):
                c = 4 * j + 1 + m
                k = (1 + m) % 4
                p = (1 + m) % 2

                @pl.when(c >= 4)
                def _():
                    wait_slot(k, src_too=False)
                pltpu.async_copy(rows1, accum.at[idl[k]], ssem[p], add=True)
                pltpu.make_async_copy(rows1, accum.at[idl[(k + 3) % 4]],
                                      ssem[1 - p]).wait()

                @pl.when(c + 3 < ch)
                def _():
                    pltpu.async_copy(
                        dst_h.at[cid, pl.ds(tbase + (c + 3) * C, C)],
                        idl[(k + 3) % 4], islot[(k + 3) % 4])
            return 0

        lax.fori_loop(0, (ch - 1) // 4, pbody2, 0)
        pltpu.make_async_copy(rows1, accum.at[idl[0]], ss0).wait()
        plsc.subcore_barrier()
        copy_out(cnt_out)

    return body(x_all, src_stack, dst_stack, z128, o128)


def _tc_dense(summed, cnt, x_all, wl, bl, wr, g, b):
    _, n, d = summed.shape
    bs = 1000
    nb = n // bs

    def body(s_ref, c_ref, x_ref, wl_ref, bl_ref, wr_ref, g_ref, b_ref, o_ref):
        mean = s_ref[0] / jnp.maximum(c_ref[0], 1.0)
        x = x_ref[...]
        h = (jnp.dot(mean, wl_ref[0], preferred_element_type=jnp.float32,
                     precision=lax.Precision.HIGHEST)
             + bl_ref[0]
             + jnp.dot(x, wr_ref[0], preferred_element_type=jnp.float32,
                       precision=lax.Precision.HIGHEST))
        t = h + x
        mu = jnp.mean(t, axis=1, keepdims=True)
        var = jnp.mean(jnp.square(t - mu), axis=1, keepdims=True)
        o_ref[0] = (t - mu) * lax.rsqrt(var + 1e-5) * g_ref[0] + b_ref[0]

    return pl.pallas_call(
        body,
        grid=(2, nb),
        in_specs=[
            pl.BlockSpec((1, bs, d), lambda i, j: (i, j, 0)),
            pl.BlockSpec((1, bs, d), lambda i, j: (i, j, 0)),
            pl.BlockSpec((bs, d), lambda i, j: ((1 - i) * nb + j, 0)),
            pl.BlockSpec((1, d, d), lambda i, j: (i, 0, 0)),
            pl.BlockSpec((1, 1, d), lambda i, j: (i, 0, 0)),
            pl.BlockSpec((1, d, d), lambda i, j: (i, 0, 0)),
            pl.BlockSpec((1, 1, d), lambda i, j: (i, 0, 0)),
            pl.BlockSpec((1, 1, d), lambda i, j: (i, 0, 0)),
        ],
        out_specs=pl.BlockSpec((1, bs, d), lambda i, j: (i, j, 0)),
        out_shape=jax.ShapeDtypeStruct((2, n, d), jnp.float32),
    )(summed, cnt, x_all, wl, bl, wr, g, b)


def kernel(x_sample, x_gene, edge_index_expresses, edge_index_expressed_by,
           W_l_sg, b_l_sg, W_r_sg, W_l_gs, b_l_gs, W_r_gs,
           ln_g_sample, ln_b_sample, ln_g_gene, ln_b_gene):
    n_s, d = x_sample.shape
    n_g = x_gene.shape[0]
    e = edge_index_expresses.shape[1]
    assert n_s == n_g and n_s % 8 == 0

    src0 = edge_index_expressed_by[0].astype(jnp.int32)
    dst0 = edge_index_expressed_by[1].astype(jnp.int32)
    src1 = edge_index_expresses[0].astype(jnp.int32) + n_g
    dst1 = edge_index_expresses[1].astype(jnp.int32)
    x_all = jnp.concatenate([x_gene, x_sample], axis=0)

    ch = -(-e // (NS * C))
    while ch % 4 != 1:
        ch += 1
    epad = NS * ch * C
    trash = n_s
    pad = epad - e
    src_stack = jnp.stack([
        jnp.pad(src0, (0, pad)),
        jnp.pad(src1, (0, pad), constant_values=n_g),
    ])
    dst_stack = jnp.stack([
        jnp.pad(dst0, (0, pad), constant_values=trash),
        jnp.pad(dst1, (0, pad), constant_values=trash),
    ])
    z128 = jnp.zeros((C, d), jnp.float32)
    o128 = jnp.ones((C, d), jnp.float32)

    summed, cnt = _sc_aggregate(x_all, src_stack, dst_stack, z128, o128,
                                n_s, d, ch)

    wl = jnp.stack([W_l_gs, W_l_sg])
    bl = jnp.stack([b_l_gs, b_l_sg])[:, None, :]
    wr = jnp.stack([W_r_gs, W_r_sg])
    g = jnp.stack([ln_g_sample, ln_g_gene])[:, None, :]
    b = jnp.stack([ln_b_sample, ln_b_gene])[:, None, :]
    return _tc_dense(summed, cnt, x_all, wl, bl, wr, g, b)
